# baseline (speedup 1.0000x reference)
import functools

import jax
import jax.numpy as jnp
from jax import lax
from jax.experimental import pallas as pl
from jax.experimental.pallas import tpu as pltpu
from jax.experimental.pallas import tpu_sc as plsc

P = 64
OUT = 128
EPS = 1e-5
CH = 16


def _sc_resample(z_flat, w_flat, R):
    info = plsc.get_sparse_core_info()
    nw = info.num_cores * info.num_subcores
    rays_per_w = R // nw
    n_chunks = rays_per_w // CH
    n_half = n_chunks // 2

    mesh = plsc.VectorSubcoreMesh(core_axis_name="c", subcore_axis_name="s")

    scratch = dict(
        zbuf0=pltpu.VMEM((CH * P + 16,), jnp.float32),
        wbuf0=pltpu.VMEM((CH * P + 16,), jnp.float32),
        zbuf1=pltpu.VMEM((CH * P + 16,), jnp.float32),
        wbuf1=pltpu.VMEM((CH * P + 16,), jnp.float32),
        obuf0=pltpu.VMEM((CH * OUT,), jnp.float32),
        obuf1=pltpu.VMEM((CH * OUT,), jnp.float32),
        cdf_a=pltpu.VMEM((80,), jnp.float32),
        bins_a=pltpu.VMEM((80,), jnp.float32),
        hist_a=pltpu.VMEM((80,), jnp.float32),
        cdf_b=pltpu.VMEM((80,), jnp.float32),
        bins_b=pltpu.VMEM((80,), jnp.float32),
        hist_b=pltpu.VMEM((80,), jnp.float32),
        cdf_c=pltpu.VMEM((80,), jnp.float32),
        bins_c=pltpu.VMEM((80,), jnp.float32),
        hist_c=pltpu.VMEM((80,), jnp.float32),
        cdf_d=pltpu.VMEM((80,), jnp.float32),
        bins_d=pltpu.VMEM((80,), jnp.float32),
        hist_d=pltpu.VMEM((80,), jnp.float32),
        sz0=pltpu.SemaphoreType.DMA,
        sw0=pltpu.SemaphoreType.DMA,
        sz1=pltpu.SemaphoreType.DMA,
        sw1=pltpu.SemaphoreType.DMA,
        so0=pltpu.SemaphoreType.DMA,
        so1=pltpu.SemaphoreType.DMA,
    )

    @functools.partial(
        pl.kernel,
        out_type=jax.ShapeDtypeStruct((R * OUT,), jnp.float32),
        mesh=mesh,
        compiler_params=pltpu.CompilerParams(
            use_tc_tiling_on_sc=False, needs_layout_passes=False),
        scratch_types=list(scratch.values()),
    )
    def sc_kernel(z_hbm, w_hbm, out_hbm,
                  zbuf0, wbuf0, zbuf1, wbuf1, obuf0, obuf1,
                  cdf_a, bins_a, hist_a, cdf_b, bins_b, hist_b,
                  cdf_c, bins_c, hist_c, cdf_d, bins_d, hist_d,
                  sz0, sw0, sz1, sw1, so0, so1):
        cid = lax.axis_index("c")
        sid = lax.axis_index("s")
        wid = sid * info.num_cores + cid
        row_base = wid * rays_per_w
        lanes = lax.iota(jnp.int32, 16)
        lanes_f = lanes.astype(jnp.float32)
        zero16 = lanes_f * 0.0
        ones16 = zero16 + 1.0
        last16 = lanes * 0 + 15
        gmask = lanes < 14
        cdfs = [cdf_a, cdf_b, cdf_c, cdf_d]
        binss = [bins_a, bins_b, bins_c, bins_d]
        hists = [hist_a, hist_b, hist_c, hist_d]
        for k in range(4):
            cdfs[k][pl.ds(0, 16)] = zero16
            for c in range(4):
                hists[k][pl.ds(16 * c, 16)] = zero16
        zbuf0[pl.ds(CH * P, 16)] = zero16
        zbuf1[pl.ds(CH * P, 16)] = zero16
        wbuf0[pl.ds(CH * P, 16)] = zero16
        wbuf1[pl.ds(CH * P, 16)] = zero16
        u = [(lanes_f + (16.0 * c)) * (1.0 / 63.0) for c in range(4)]

        gdn = lax.GatherDimensionNumbers(
            offset_dims=(), collapsed_slice_dims=(0,), start_index_map=(0,))

        def vperm(v, idx):
            return lax.gather(v, idx[:, None], gdn, (1,),
                              mode=lax.GatherScatterMode.PROMISE_IN_BOUNDS)

        def bcast_last(v):
            return vperm(v, last16)

        perm_idx = {d: lanes ^ d for d in (8, 4, 2, 1)}
        perm_lo = {d: (lanes & d) == 0 for d in (8, 4, 2, 1)}

        def bitonic_clean(x):
            for d in (8, 4, 2, 1):
                p = vperm(x, perm_idx[d])
                x = jnp.where(perm_lo[d], jnp.minimum(x, p), jnp.maximum(x, p))
            return x

        def issue_in(ci, zb, wb, sz, sw):
            base = (row_base + ci * CH) * P
            pltpu.async_copy(z_hbm.at[pl.ds(base, CH * P)], zb.at[pl.ds(0, CH * P)], sz)
            pltpu.async_copy(w_hbm.at[pl.ds(base, CH * P)], wb.at[pl.ds(0, CH * P)], sw)

        def wait_in(zb, wb, sz, sw):
            pltpu.make_async_copy(z_hbm.at[pl.ds(0, CH * P)], zb.at[pl.ds(0, CH * P)], sz).wait()
            pltpu.make_async_copy(w_hbm.at[pl.ds(0, CH * P)], wb.at[pl.ds(0, CH * P)], sw).wait()

        def issue_out(ci, ob, so):
            base = (row_base + ci * CH) * OUT
            pltpu.async_copy(ob, out_hbm.at[pl.ds(base, CH * OUT)], so)

        def wait_out(ob, so):
            pltpu.make_async_copy(ob, out_hbm.at[pl.ds(0, CH * OUT)], so).wait()

        def ray(zbuf, wbuf, obuf, cdf_arr, bins_arr, hist, r):
            zb = r * P
            ob = r * OUT
            zc = []
            for c in range(4):
                zv = zbuf[pl.ds(zb + 16 * c, 16)]
                zs = plsc.load_gather(zbuf, [lanes + (zb + 16 * c + 1)])
                zc.append(zv)
                bins_arr[pl.ds(16 * c, 16)] = 0.5 * (zv + zs)
            carry = zero16
            cums = []
            for c in range(4):
                wv = plsc.load_gather(wbuf, [lanes + (zb + 1 + 16 * c)]) + EPS
                if c == 3:
                    wv = jnp.where(gmask, wv, 0.0)
                cs = plsc.cumsum(wv) + carry
                cums.append(cs)
                carry = bcast_last(cs)
            inv = ones16 / carry
            ms = []
            for c in range(4):
                cn = cums[c] * inv
                plsc.store_scatter(cdf_arr, [lanes + (1 + 16 * c)], cn)
                x = cn * 63.0
                xi = x.astype(jnp.int32)
                xi = xi + jnp.where(xi.astype(jnp.float32) < x, 1, 0)
                xi = jnp.minimum(xi, 63)
                ms.append(xi)
            for c in range(4):
                plsc.addupdate_scatter(hist, [ms[c]], ones16,
                                       mask=gmask if c == 3 else None)
            hcarry = ones16
            sample = []
            for c in range(4):
                hv = hist[pl.ds(16 * c, 16)]
                hist[pl.ds(16 * c, 16)] = zero16
                fin = plsc.cumsum(hv) + hcarry
                hcarry = bcast_last(fin)
                inds = fin.astype(jnp.int32)
                below = inds - 1
                above = jnp.minimum(inds, 62)
                cg0 = plsc.load_gather(cdf_arr, [below])
                cg1 = plsc.load_gather(cdf_arr, [above])
                bg0 = plsc.load_gather(bins_arr, [below])
                bg1 = plsc.load_gather(bins_arr, [above])
                denom = cg1 - cg0
                denom = jnp.where(denom < EPS, 1.0, denom)
                t = (u[c] - cg0) / denom
                sample.append(bg0 + t * (bg1 - bg0))
            bv = [lax.rev(sample[3 - c], (0,)) for c in range(4)]
            lo = [jnp.minimum(zc[i], bv[i]) for i in range(4)]
            hi = [jnp.maximum(zc[i], bv[i]) for i in range(4)]
            a0 = jnp.minimum(lo[0], lo[2]); a2 = jnp.maximum(lo[0], lo[2])
            a1 = jnp.minimum(lo[1], lo[3]); a3 = jnp.maximum(lo[1], lo[3])
            b0 = jnp.minimum(hi[0], hi[2]); b2 = jnp.maximum(hi[0], hi[2])
            b1 = jnp.minimum(hi[1], hi[3]); b3 = jnp.maximum(hi[1], hi[3])
            out8 = [
                jnp.minimum(a0, a1), jnp.maximum(a0, a1),
                jnp.minimum(a2, a3), jnp.maximum(a2, a3),
                jnp.minimum(b0, b1), jnp.maximum(b0, b1),
                jnp.minimum(b2, b3), jnp.maximum(b2, b3),
            ]
            for i in range(8):
                obuf[pl.ds(ob + 16 * i, 16)] = bitonic_clean(out8[i])

        def process(zbuf, wbuf, obuf):
            def body(i, _):
                for k in range(4):
                    ray(zbuf, wbuf, obuf, cdfs[k], binss[k], hists[k], 4 * i + k)
                return 0
            lax.fori_loop(0, CH // 4, body, 0)

        issue_in(0, zbuf0, wbuf0, sz0, sw0)

        def chunk_body(i, _):
            issue_in(2 * i + 1, zbuf1, wbuf1, sz1, sw1)
            wait_in(zbuf0, wbuf0, sz0, sw0)

            @pl.when(i > 0)
            def _():
                wait_out(obuf0, so0)

            process(zbuf0, wbuf0, obuf0)
            issue_out(2 * i, obuf0, so0)

            @pl.when(i + 1 < n_half)
            def _():
                issue_in(2 * i + 2, zbuf0, wbuf0, sz0, sw0)

            wait_in(zbuf1, wbuf1, sz1, sw1)

            @pl.when(i > 0)
            def _():
                wait_out(obuf1, so1)

            process(zbuf1, wbuf1, obuf1)
            issue_out(2 * i + 1, obuf1, so1)
            return 0

        lax.fori_loop(0, n_half, chunk_body, 0)
        wait_out(obuf0, so0)
        wait_out(obuf1, so1)

    return sc_kernel(z_flat, w_flat)


def kernel(origins, directions, lengths, ray_weights, xys):
    B, R, _ = lengths.shape
    z_flat = lengths.reshape(B * R * P)
    w_flat = ray_weights.reshape(B * R * P)
    out = _sc_resample(z_flat, w_flat, B * R)
    z_vals = out.reshape(B, R, OUT)
    return (origins, directions, z_vals, xys)


# speedup vs baseline: 1.0834x; 1.0834x over previous
import functools

import jax
import jax.numpy as jnp
from jax import lax
from jax.experimental import pallas as pl
from jax.experimental.pallas import tpu as pltpu
from jax.experimental.pallas import tpu_sc as plsc

P = 64
OUT = 128
EPS = 1e-5
CH = 32


def _sc_resample(z_flat, w_flat, R):
    info = plsc.get_sparse_core_info()
    nw = info.num_cores * info.num_subcores
    rays_per_w = R // nw
    n_chunks = rays_per_w // CH
    n_half = n_chunks // 2

    mesh = plsc.VectorSubcoreMesh(core_axis_name="c", subcore_axis_name="s")

    scratch = dict(
        zbuf0=pltpu.VMEM((CH * P + 16,), jnp.float32),
        wbuf0=pltpu.VMEM((CH * P + 16,), jnp.float32),
        zbuf1=pltpu.VMEM((CH * P + 16,), jnp.float32),
        wbuf1=pltpu.VMEM((CH * P + 16,), jnp.float32),
        obuf0=pltpu.VMEM((CH * OUT,), jnp.float32),
        obuf1=pltpu.VMEM((CH * OUT,), jnp.float32),
        cdf_a=pltpu.VMEM((80,), jnp.float32),
        bins_a=pltpu.VMEM((80,), jnp.float32),
        hist_a=pltpu.VMEM((80,), jnp.float32),
        cdf_b=pltpu.VMEM((80,), jnp.float32),
        bins_b=pltpu.VMEM((80,), jnp.float32),
        hist_b=pltpu.VMEM((80,), jnp.float32),
        cdf_c=pltpu.VMEM((80,), jnp.float32),
        bins_c=pltpu.VMEM((80,), jnp.float32),
        hist_c=pltpu.VMEM((80,), jnp.float32),
        cdf_d=pltpu.VMEM((80,), jnp.float32),
        bins_d=pltpu.VMEM((80,), jnp.float32),
        hist_d=pltpu.VMEM((80,), jnp.float32),
        sz0=pltpu.SemaphoreType.DMA,
        sw0=pltpu.SemaphoreType.DMA,
        sz1=pltpu.SemaphoreType.DMA,
        sw1=pltpu.SemaphoreType.DMA,
        so0=pltpu.SemaphoreType.DMA,
        so1=pltpu.SemaphoreType.DMA,
    )

    @functools.partial(
        pl.kernel,
        out_type=jax.ShapeDtypeStruct((R * OUT,), jnp.float32),
        mesh=mesh,
        compiler_params=pltpu.CompilerParams(
            use_tc_tiling_on_sc=False, needs_layout_passes=False),
        scratch_types=list(scratch.values()),
    )
    def sc_kernel(z_hbm, w_hbm, out_hbm,
                  zbuf0, wbuf0, zbuf1, wbuf1, obuf0, obuf1,
                  cdf_a, bins_a, hist_a, cdf_b, bins_b, hist_b,
                  cdf_c, bins_c, hist_c, cdf_d, bins_d, hist_d,
                  sz0, sw0, sz1, sw1, so0, so1):
        cid = lax.axis_index("c")
        sid = lax.axis_index("s")
        wid = sid * info.num_cores + cid
        row_base = wid * rays_per_w
        lanes = lax.iota(jnp.int32, 16)
        lanes_f = lanes.astype(jnp.float32)
        zero16 = lanes_f * 0.0
        ones16 = zero16 + 1.0
        last16 = lanes * 0 + 15
        gmask = lanes < 14
        cdfs = [cdf_a, cdf_b, cdf_c, cdf_d]
        binss = [bins_a, bins_b, bins_c, bins_d]
        hists = [hist_a, hist_b, hist_c, hist_d]
        for k in range(4):
            cdfs[k][pl.ds(0, 16)] = zero16
            for c in range(4):
                hists[k][pl.ds(16 * c, 16)] = zero16
        zbuf0[pl.ds(CH * P, 16)] = zero16
        zbuf1[pl.ds(CH * P, 16)] = zero16
        wbuf0[pl.ds(CH * P, 16)] = zero16
        wbuf1[pl.ds(CH * P, 16)] = zero16
        u = [(lanes_f + (16.0 * c)) * (1.0 / 63.0) for c in range(4)]

        gdn = lax.GatherDimensionNumbers(
            offset_dims=(), collapsed_slice_dims=(0,), start_index_map=(0,))

        def vperm(v, idx):
            return lax.gather(v, idx[:, None], gdn, (1,),
                              mode=lax.GatherScatterMode.PROMISE_IN_BOUNDS)

        def bcast_last(v):
            return vperm(v, last16)

        def issue_in(ci, zb, wb, sz, sw):
            base = (row_base + ci * CH) * P
            pltpu.async_copy(z_hbm.at[pl.ds(base, CH * P)], zb.at[pl.ds(0, CH * P)], sz)
            pltpu.async_copy(w_hbm.at[pl.ds(base, CH * P)], wb.at[pl.ds(0, CH * P)], sw)

        def wait_in(zb, wb, sz, sw):
            pltpu.make_async_copy(z_hbm.at[pl.ds(0, CH * P)], zb.at[pl.ds(0, CH * P)], sz).wait()
            pltpu.make_async_copy(w_hbm.at[pl.ds(0, CH * P)], wb.at[pl.ds(0, CH * P)], sw).wait()

        def issue_out(ci, ob, so):
            base = (row_base + ci * CH) * OUT
            pltpu.async_copy(ob, out_hbm.at[pl.ds(base, CH * OUT)], so)

        def wait_out(ob, so):
            pltpu.make_async_copy(ob, out_hbm.at[pl.ds(0, CH * OUT)], so).wait()

        def ray(zbuf, wbuf, obuf, cdf_arr, bins_arr, hist, r):
            zb = r * P
            ob = r * OUT
            zc = []
            for c in range(4):
                zv = zbuf[pl.ds(zb + 16 * c, 16)]
                zs = plsc.load_gather(zbuf, [lanes + (zb + 16 * c + 1)])
                zc.append(zv)
                bins_arr[pl.ds(16 * c, 16)] = 0.5 * (zv + zs)
            carry = zero16
            cums = []
            for c in range(4):
                wv = plsc.load_gather(wbuf, [lanes + (zb + 1 + 16 * c)]) + EPS
                if c == 3:
                    wv = jnp.where(gmask, wv, 0.0)
                cs = plsc.cumsum(wv) + carry
                cums.append(cs)
                carry = bcast_last(cs)
            inv = ones16 / carry
            ms = []
            for c in range(4):
                cn = cums[c] * inv
                plsc.store_scatter(cdf_arr, [lanes + (1 + 16 * c)], cn)
                x = cn * 63.0
                xi = x.astype(jnp.int32)
                xi = xi + jnp.where(xi.astype(jnp.float32) < x, 1, 0)
                xi = jnp.minimum(xi, 63)
                ms.append(xi)
            for c in range(4):
                plsc.addupdate_scatter(hist, [ms[c]], ones16,
                                       mask=gmask if c == 3 else None)
            hcarry = ones16
            sample = []
            for c in range(4):
                hv = hist[pl.ds(16 * c, 16)]
                hist[pl.ds(16 * c, 16)] = zero16
                fin = plsc.cumsum(hv) + hcarry
                hcarry = bcast_last(fin)
                inds = fin.astype(jnp.int32)
                below = inds - 1
                above = inds
                cg0 = plsc.load_gather(cdf_arr, [below])
                cg1 = plsc.load_gather(cdf_arr, [above])
                bg0 = plsc.load_gather(bins_arr, [below])
                bg1 = plsc.load_gather(bins_arr, [above])
                denom = cg1 - cg0
                denom = jnp.where(denom < EPS, 1.0, denom)
                t = (u[c] - cg0) / denom
                sample.append(bg0 + t * (bg1 - bg0))
            bv = [lax.rev(sample[3 - c], (0,)) for c in range(4)]
            lo = [jnp.minimum(zc[i], bv[i]) for i in range(4)]
            hi = [jnp.maximum(zc[i], bv[i]) for i in range(4)]
            a0 = jnp.minimum(lo[0], lo[2]); a2 = jnp.maximum(lo[0], lo[2])
            a1 = jnp.minimum(lo[1], lo[3]); a3 = jnp.maximum(lo[1], lo[3])
            b0 = jnp.minimum(hi[0], hi[2]); b2 = jnp.maximum(hi[0], hi[2])
            b1 = jnp.minimum(hi[1], hi[3]); b3 = jnp.maximum(hi[1], hi[3])
            out8 = [
                jnp.minimum(a0, a1), jnp.maximum(a0, a1),
                jnp.minimum(a2, a3), jnp.maximum(a2, a3),
                jnp.minimum(b0, b1), jnp.maximum(b0, b1),
                jnp.minimum(b2, b3), jnp.maximum(b2, b3),
            ]
            for i in range(8):
                obuf[pl.ds(ob + 16 * i, 16)] = jnp.sort(out8[i])

        def process(zbuf, wbuf, obuf):
            def body(i, _):
                for k in range(4):
                    ray(zbuf, wbuf, obuf, cdfs[k], binss[k], hists[k], 4 * i + k)
                return 0
            lax.fori_loop(0, CH // 4, body, 0)

        issue_in(0, zbuf0, wbuf0, sz0, sw0)

        def chunk_body(i, _):
            issue_in(2 * i + 1, zbuf1, wbuf1, sz1, sw1)
            wait_in(zbuf0, wbuf0, sz0, sw0)

            @pl.when(i > 0)
            def _():
                wait_out(obuf0, so0)

            process(zbuf0, wbuf0, obuf0)
            issue_out(2 * i, obuf0, so0)

            @pl.when(i + 1 < n_half)
            def _():
                issue_in(2 * i + 2, zbuf0, wbuf0, sz0, sw0)

            wait_in(zbuf1, wbuf1, sz1, sw1)

            @pl.when(i > 0)
            def _():
                wait_out(obuf1, so1)

            process(zbuf1, wbuf1, obuf1)
            issue_out(2 * i + 1, obuf1, so1)
            return 0

        lax.fori_loop(0, n_half, chunk_body, 0)
        wait_out(obuf0, so0)
        wait_out(obuf1, so1)

    return sc_kernel(z_flat, w_flat)


def kernel(origins, directions, lengths, ray_weights, xys):
    B, R, _ = lengths.shape
    z_flat = lengths.reshape(B * R * P)
    w_flat = ray_weights.reshape(B * R * P)
    out = _sc_resample(z_flat, w_flat, B * R)
    z_vals = out.reshape(B, R, OUT)
    return (origins, directions, z_vals, xys)
